# baseline (speedup 1.0000x reference)
import functools

import jax
import jax.numpy as jnp
from jax import lax
from jax.experimental import pallas as pl
from jax.experimental.pallas import tpu as pltpu
from jax.experimental.pallas import tpu_sc as plsc

_NW = 32
_TILE = 32768


def kernel(values, indices):
    n = values.shape[0]
    nfull = n // _TILE
    rem = n - nfull * _TILE
    kmax = (nfull + _NW - 1) // _NW
    mesh = plsc.VectorSubcoreMesh(core_axis_name="c", subcore_axis_name="s")

    @functools.partial(
        pl.kernel,
        mesh=mesh,
        out_type=jax.ShapeDtypeStruct((n,), jnp.float32),
        scratch_types=[
            pltpu.VMEM((_TILE,), jnp.float32),
            pltpu.VMEM((_TILE,), jnp.float32),
            pltpu.VMEM((max(rem, 1),), jnp.float32),
            pltpu.SemaphoreType.DMA,
            pltpu.SemaphoreType.DMA,
            pltpu.SemaphoreType.DMA,
            pltpu.SemaphoreType.DMA,
            pltpu.SemaphoreType.DMA,
        ],
    )
    def sc_copy(v_hbm, o_hbm, b0, b1, tail_buf, r0, r1, w0, w1, tsem):
        cid = lax.axis_index("c")
        sid = lax.axis_index("s")
        wid = sid * 2 + cid

        bufs = (b0, b1)
        rsems = (r0, r1)
        wsems = (w0, w1)

        def rd(k):
            off = (wid + k * _NW) * _TILE
            return pltpu.make_async_copy(
                v_hbm.at[pl.ds(off, _TILE)], bufs[k % 2], rsems[k % 2])

        def wr(k):
            off = (wid + k * _NW) * _TILE
            return pltpu.make_async_copy(
                bufs[k % 2], o_hbm.at[pl.ds(off, _TILE)], wsems[k % 2])

        def valid(k):
            return wid + k * _NW < nfull

        tail_rd = pltpu.make_async_copy(
            v_hbm.at[pl.ds(nfull * _TILE, max(rem, 1))], tail_buf, tsem)
        tail_wr = pltpu.make_async_copy(
            tail_buf, o_hbm.at[pl.ds(nfull * _TILE, max(rem, 1))], tsem)

        if rem:
            @pl.when(wid == _NW - 1)
            def _tail_start():
                tail_rd.start()

        for k in range(min(2, kmax)):
            @pl.when(valid(k))
            def _prime(k=k):
                rd(k).start()

        for k in range(kmax):
            @pl.when(valid(k))
            def _write_cur(k=k):
                rd(k).wait()
                wr(k).start()

            if k + 2 < kmax:
                @pl.when(valid(k + 2))
                def _read_ahead(k=k):
                    wr(k).wait()
                    rd(k + 2).start()

        for k in range(kmax):
            late = valid(k + 2) if k + 2 < kmax else False
            @pl.when(valid(k) & jnp.logical_not(late))
            def _final_wait(k=k):
                wr(k).wait()

        if rem:
            @pl.when(wid == _NW - 1)
            def _tail_finish():
                tail_rd.wait()
                tail_wr.start()
                tail_wr.wait()

    return sc_copy(values)


# speedup vs baseline: 3.0564x; 3.0564x over previous
import jax
import jax.numpy as jnp
from jax.experimental import pallas as pl

_BLOCK = 1536 * 1024


def _copy_block(v_ref, o_ref):
    o_ref[...] = v_ref[...]


def kernel(values, indices):
    n = values.shape[0]
    grid = (pl.cdiv(n, _BLOCK),)
    return pl.pallas_call(
        _copy_block,
        grid=grid,
        in_specs=[pl.BlockSpec((_BLOCK,), lambda i: (i,))],
        out_specs=pl.BlockSpec((_BLOCK,), lambda i: (i,)),
        out_shape=jax.ShapeDtypeStruct(values.shape, values.dtype),
    )(values)
